# baseline (speedup 1.0000x reference)
import jax
import jax.numpy as jnp
from jax import lax
from jax.experimental import pallas as pl
from jax.experimental.pallas import tpu as pltpu
from jax.experimental.pallas import tpu_sc as plsc

N = 10000
E = 320000
D = 128
DH = D // 2
NRELS = 8
NTYPES = 4

NC = 2
NS = 16

EP = 327680
ET = EP // NS
C1 = 1024
C2 = 64
NTRASH = 8
NPK = N * NRELS


def _k1_body(f_ref, w_ref, y_ref):
    y_ref[...] = jnp.dot(f_ref[...], w_ref[0], preferred_element_type=jnp.float32)


def _edge_transform(feats, W_edge):
    BN = 2000
    NB = N // BN
    return pl.pallas_call(
        _k1_body,
        grid=(NRELS, NB),
        in_specs=[
            pl.BlockSpec((BN, D), lambda r, i: (i, 0)),
            pl.BlockSpec((1, D, D), lambda r, i: (r, 0, 0)),
        ],
        out_specs=pl.BlockSpec((BN, D), lambda r, i: (r * NB + i, 0)),
        out_shape=jax.ShapeDtypeStruct((NRELS * N, D), jnp.float32),
    )(feats, W_edge)


def _k2_body(epk_hbm, y_hbm,
             agg_hbm, cnt_hbm,
             eb1, onesb, pkb1,
             eb20, k2b0, pkb0, dsb0, sclb0, rows0, rowsh0, gs0, cs0,
             eb21, k2b1, pkb1b, dsb1, sclb1, rows1, rowsh1, gs1, cs1,
             zb1,
             counts_sp, agg_sp):
    c = lax.axis_index("c")
    s = lax.axis_index("s")

    def fill_z1(i, _):
        zb1[pl.ds(i * 16, 16)] = jnp.zeros((16,), jnp.float32)
        return 0

    lax.fori_loop(0, 5008 // 16, fill_z1, 0)

    def fill_z2(i, _):
        for j in range(DH // 16):
            rowsh0[i, pl.ds(j * 16, 16)] = jnp.zeros((16,), jnp.float32)
        return 0

    lax.fori_loop(0, C2, fill_z2, 0)

    def fill_ones(i, _):
        onesb[pl.ds(i * 16, 16)] = jnp.ones((16,), jnp.float32)
        return 0

    lax.fori_loop(0, C1 // 16, fill_ones, 0)

    @pl.when(s < NS - 1)
    def _():
        pltpu.sync_copy(zb1, counts_sp.at[pl.ds(s * 5008, 5008)])
        for j in range(9):
            pltpu.sync_copy(rowsh0, agg_sp.at[pl.ds(s * 624 + j * 64, 64)])
        pltpu.sync_copy(rowsh0.at[pl.ds(0, 48)],
                        agg_sp.at[pl.ds(s * 624 + 576, 48)])

    @pl.when(s == NS - 1)
    def _():
        pltpu.sync_copy(zb1.at[pl.ds(0, 4896)],
                        counts_sp.at[pl.ds(15 * 5008, 4896)])
        for j in range(10):
            pltpu.sync_copy(rowsh0, agg_sp.at[pl.ds(15 * 624 + j * 64, 64)])
        pltpu.sync_copy(rowsh0.at[pl.ds(0, NTRASH)],
                        agg_sp.at[pl.ds(N, NTRASH)])

    plsc.subcore_barrier()

    def count_chunk(k, _):
        base = (s * ET + k * C1) * 3
        pltpu.sync_copy(epk_hbm.at[pl.ds(base, C1 * 3)], eb1)

        def mk_pk(i, _):
            blk = (i // 4) * 192
            off = pl.ds(blk + 64 + (i % 4) * 16, 16)
            off2 = pl.ds(blk + 128 + (i % 4) * 16, 16)
            pkb1[pl.ds(i * 16, 16)] = eb1[off] * NRELS + eb1[off2]
            return 0

        lax.fori_loop(0, C1 // 16, mk_pk, 0)
        pltpu.sync_copy(onesb, counts_sp.at[pkb1], add=True)
        return 0

    lax.fori_loop(0, ET // C1, count_chunk, 0)
    plsc.subcore_barrier()

    @pl.when(s < NS - 1)
    def _():
        cs = pl.ds(s * 5008, 5008)
        pltpu.sync_copy(counts_sp.at[cs], zb1)

        def recip(i, _):
            off = pl.ds(i * 16, 16)
            zb1[off] = 1.0 / zb1[off]
            return 0

        lax.fori_loop(0, 5008 // 16, recip, 0)
        pltpu.sync_copy(zb1, counts_sp.at[cs])

    @pl.when(s == NS - 1)
    def _():
        cs = pl.ds(15 * 5008, 4896)
        zs = pl.ds(0, 4896)
        pltpu.sync_copy(counts_sp.at[cs], zb1.at[zs])

        def recip(i, _):
            off = pl.ds(i * 16, 16)
            zb1[off] = 1.0 / zb1[off]
            return 0

        lax.fori_loop(0, 4896 // 16, recip, 0)
        pltpu.sync_copy(zb1.at[zs], counts_sp.at[cs])
    plsc.subcore_barrier()

    sets = ((eb20, k2b0, pkb0, dsb0, sclb0, rows0, rowsh0, gs0, cs0),
            (eb21, k2b1, pkb1b, dsb1, sclb1, rows1, rowsh1, gs1, cs1))

    def ek_load(kidx, eb, k2, pk, ds_):
        base = (s * ET + kidx * C2) * 3
        pltpu.sync_copy(epk_hbm.at[pl.ds(base, C2 * 3)], eb)

        def mk_keys(i, _):
            off = pl.ds(i * 16, 16)
            off2 = pl.ds(64 + i * 16, 16)
            off3 = pl.ds(128 + i * 16, 16)
            et = eb[off3]
            k2[off] = et * N + eb[off]
            pk[off] = eb[off2] * NRELS + et
            ds_[off] = eb[off2]
            return 0

        lax.fori_loop(0, C2 // 16, mk_keys, 0)

    def g_start(k2, pk, rows, scl, gs, cs):
        dy = pltpu.async_copy(y_hbm.at[k2], rows, gs)
        dc = pltpu.async_copy(counts_sp.at[pk], scl, cs)
        return dy, dc

    def scale_scatter(rows, rowsh, scl, ds_):
        def scale_row(i, _):
            sv = plsc.load_gather(scl, [jnp.full((16,), i, jnp.int32)])
            for j in range(DH // 16):
                rowsh[i, pl.ds(j * 16, 16)] = (
                    rows[i, pl.ds(c * DH + j * 16, 16)] * sv)
            return 0

        lax.fori_loop(0, C2, scale_row, 0, unroll=4)
        pltpu.sync_copy(rowsh, agg_sp.at[ds_], add=True)

    def t_ek(kidx, t):
        ek_load(kidx, t[0], t[1], t[2], t[3])

    def t_gstart(t):
        return g_start(t[1], t[2], t[5], t[4], t[7], t[8])

    def t_ss(t):
        scale_scatter(t[5], t[6], t[4], t[3])

    def one_chunk(k, _):
        t_ek(k, sets[0])
        dca = pltpu.async_copy(counts_sp.at[sets[0][2]], sets[0][4], sets[0][8])
        dya = pltpu.async_copy(y_hbm.at[sets[0][1]], sets[0][5], sets[0][7])
        dya.wait()
        dca.wait()
        t_ss(sets[0])
        return 0

    lax.fori_loop(0, ET // C2, one_chunk, 0)
    plsc.subcore_barrier()

    @pl.when(s < NS - 1)
    def _():
        for off in range(0, 576, 64):
            sp = pl.ds(s * 624 + off, 64)
            pltpu.sync_copy(agg_sp.at[sp], rowsh0)
            pltpu.sync_copy(rowsh0, agg_hbm.at[c, sp])
        sp = pl.ds(s * 624 + 576, 48)
        pltpu.sync_copy(agg_sp.at[sp], rowsh0.at[pl.ds(0, 48)])
        pltpu.sync_copy(rowsh0.at[pl.ds(0, 48)], agg_hbm.at[c, sp])

    @pl.when(s == NS - 1)
    def _():
        for off in range(0, 640, 64):
            sp = pl.ds(15 * 624 + off, 64)
            pltpu.sync_copy(agg_sp.at[sp], rowsh0)
            pltpu.sync_copy(rowsh0, agg_hbm.at[c, sp])

    @pl.when(jnp.logical_and(c == 0, s < NS - 1))
    def _():
        cs = pl.ds(s * 5008, 5008)
        pltpu.sync_copy(counts_sp.at[cs], zb1)
        pltpu.sync_copy(zb1, cnt_hbm.at[cs])

    @pl.when(jnp.logical_and(c == 0, s == NS - 1))
    def _():
        cs = pl.ds(15 * 5008, 4880)
        pltpu.sync_copy(counts_sp.at[cs], zb1.at[pl.ds(0, 4880)])
        pltpu.sync_copy(zb1.at[pl.ds(0, 4880)], cnt_hbm.at[cs])


def _sc_aggregate(epacked, y):
    mesh = plsc.VectorSubcoreMesh(core_axis_name="c", subcore_axis_name="s",
                                  num_cores=NC, num_subcores=NS)
    fn = pl.kernel(
        _k2_body,
        out_type=[
            jax.ShapeDtypeStruct((NC, N, DH), jnp.float32),
            jax.ShapeDtypeStruct((NPK,), jnp.float32),
        ],
        mesh=mesh,
        compiler_params=pltpu.CompilerParams(needs_layout_passes=False),
        scratch_types=[
            pltpu.VMEM((C1 * 3,), jnp.int32),
            pltpu.VMEM((C1,), jnp.float32),
            pltpu.VMEM((C1,), jnp.int32),
            pltpu.VMEM((C2 * 3,), jnp.int32),
            pltpu.VMEM((C2,), jnp.int32),
            pltpu.VMEM((C2,), jnp.int32),
            pltpu.VMEM((C2,), jnp.int32),
            pltpu.VMEM((C2,), jnp.float32),
            pltpu.VMEM((C2, D), jnp.float32),
            pltpu.VMEM((C2, DH), jnp.float32),
            pltpu.SemaphoreType.DMA,
            pltpu.SemaphoreType.DMA,
            pltpu.VMEM((C2 * 3,), jnp.int32),
            pltpu.VMEM((C2,), jnp.int32),
            pltpu.VMEM((C2,), jnp.int32),
            pltpu.VMEM((C2,), jnp.int32),
            pltpu.VMEM((C2,), jnp.float32),
            pltpu.VMEM((C2, D), jnp.float32),
            pltpu.VMEM((C2, DH), jnp.float32),
            pltpu.SemaphoreType.DMA,
            pltpu.SemaphoreType.DMA,
            pltpu.VMEM((5008,), jnp.float32),
            pltpu.VMEM_SHARED((NPK + 16, ), jnp.float32),
            pltpu.VMEM_SHARED((N + NTRASH, DH), jnp.float32),
        ],
    )
    return fn(epacked, y)


def _k3_body(f_ref, oh_ref, wn_ref, bn_ref, agg_ref, cnt_ref, be_ref, o_ref):
    f = f_ref[...]
    oh = oh_ref[...]
    acc = jnp.concatenate([agg_ref[0], agg_ref[1]], axis=-1)
    acc += oh @ bn_ref[...]
    nz = (cnt_ref[...] > 0.0).astype(jnp.float32)
    acc += nz @ be_ref[...]
    for t in range(NTYPES):
        acc += oh[:, t:t + 1] * jnp.dot(f, wn_ref[t], preferred_element_type=jnp.float32)
    o_ref[...] = jnp.maximum(acc, 0.0)


def _combine(feats, onehot, W_node, b_node, aggp, cnt, b_edge):
    BN = 2000
    NB = N // BN
    return pl.pallas_call(
        _k3_body,
        grid=(NB,),
        in_specs=[
            pl.BlockSpec((BN, D), lambda i: (i, 0)),
            pl.BlockSpec((BN, NTYPES), lambda i: (i, 0)),
            pl.BlockSpec((NTYPES, D, D), lambda i: (0, 0, 0)),
            pl.BlockSpec((NTYPES, D), lambda i: (0, 0)),
            pl.BlockSpec((2, BN, DH), lambda i: (0, i, 0)),
            pl.BlockSpec((BN, NRELS), lambda i: (i, 0)),
            pl.BlockSpec((NRELS, D), lambda i: (0, 0)),
        ],
        out_specs=pl.BlockSpec((BN, D), lambda i: (i, 0)),
        out_shape=jax.ShapeDtypeStruct((N, D), jnp.float32),
    )(feats, onehot, W_node, b_node, aggp, cnt, b_edge)


@jax.jit
def kernel(feats, edge_index, ntypes, etypes, W_node, b_node, W_edge, b_edge):
    npad = EP - E
    src = jnp.concatenate([edge_index[0], jnp.zeros((npad,), jnp.int32)])
    dst = jnp.concatenate([edge_index[1], jnp.full((npad,), N, jnp.int32)])
    etp = jnp.concatenate([etypes, jnp.zeros((npad,), jnp.int32)])
    epacked = jnp.stack([src.reshape(-1, 64), dst.reshape(-1, 64),
                         etp.reshape(-1, 64)], axis=1).reshape(-1)
    epacked = jnp.concatenate([epacked, jnp.zeros((C2 * 3,), jnp.int32)])
    y = _edge_transform(feats, W_edge)
    aggp, counts = _sc_aggregate(epacked, y)
    onehot = jax.nn.one_hot(ntypes, NTYPES, dtype=jnp.float32)
    cnt2d = counts.reshape(N, NRELS)
    return _combine(feats, onehot, W_node, b_node, aggp, cnt2d, b_edge)
